# baseline (speedup 1.0000x reference)
import functools

import jax
import jax.numpy as jnp
from jax import lax
from jax.experimental import pallas as pl
from jax.experimental.pallas import tpu as pltpu
from jax.experimental.pallas import tpu_sc as plsc

_NUM_WORKERS = 32
_NBUF = 4
_INIT_AHEAD = 3
_GATHER_AHEAD = 2
_POSB_W = 32


def kernel(inputs, token_table, pos_table):
    B, L = inputs.shape
    V, E = token_table.shape
    BBLK = B // _NUM_WORKERS
    nbb = BBLK // 16
    neg = E // 16

    inputs_t = jnp.swapaxes(inputs.astype(jnp.float32), 0, 1)

    mesh = plsc.VectorSubcoreMesh(core_axis_name="c", subcore_axis_name="s")

    @functools.partial(
        pl.kernel,
        mesh=mesh,
        compiler_params=pltpu.CompilerParams(use_tc_tiling_on_sc=False,
                                             needs_layout_passes=False),
        out_type=jax.ShapeDtypeStruct((L, E, B), jnp.float32),
        scratch_types=[
            pltpu.VMEM((_NBUF, BBLK), jnp.float32),
            pltpu.VMEM((_NBUF, BBLK), jnp.int32),
            pltpu.VMEM((L, E), jnp.float32),
            pltpu.VMEM_SHARED((L, _POSB_W, E), jnp.float32),
            [pltpu.VMEM((BBLK, E), jnp.float32)] * _NBUF,
            [pltpu.VMEM((E, BBLK), jnp.float32)] * _NBUF,
            pltpu.VMEM((BBLK // 16, 280), jnp.float32),
            [pltpu.SemaphoreType.DMA] * _NBUF,
            [pltpu.SemaphoreType.DMA] * _NBUF,
            [pltpu.SemaphoreType.DMA] * _NBUF,
            [pltpu.SemaphoreType.DMA] * _NBUF,
        ],
    )
    def emb_kernel(inputs_hbm, table_hbm, pos_hbm, out_hbm,
                   idxf_v, idx_v, pos_v, posb_sh, gbufs, tbufs, tpscr,
                   xsems, isems, gsems, wsems):
        sid = lax.axis_index("s")
        wid = sid * 2 + lax.axis_index("c")
        bbase = wid * BBLK

        pltpu.sync_copy(pos_hbm, pos_v)

        def build(l, carry):
            prow = [pos_v[l, pl.ds(16 * q, 16)] for q in range(neg)]

            @plsc.parallel_loop(0, _POSB_W, step=1)
            def _fill(j):
                for q in range(neg):
                    gbufs[0][j, pl.ds(16 * q, 16)] = prow[q]

            pltpu.sync_copy(gbufs[0].at[pl.ds(0, _POSB_W)], posb_sh.at[l])
            return carry

        lax.fori_loop((sid * L) // 16, ((sid + 1) * L) // 16, build, 0)
        plsc.subcore_barrier()

        def idx_cp(l, k):
            return pltpu.make_async_copy(
                inputs_hbm.at[l, pl.ds(bbase, BBLK)], idxf_v.at[k], xsems[k])

        def idx_convert(k):
            for q in range(BBLK // 16):
                idx_v[k, pl.ds(16 * q, 16)] = lax.convert_element_type(
                    idxf_v[k, pl.ds(16 * q, 16)], jnp.int32)

        def init_cps(l, k):
            return [
                pltpu.make_async_copy(
                    posb_sh.at[l],
                    gbufs[k].at[pl.ds(i * _POSB_W, _POSB_W)], isems[k])
                for i in range(BBLK // _POSB_W)
            ]

        def gather_cp(l, k):
            return pltpu.make_async_copy(
                table_hbm.at[idx_v.at[k]], gbufs[k], gsems[k])

        def wb_cp(l, k):
            return pltpu.make_async_copy(
                tbufs[k], out_hbm.at[l, :, pl.ds(bbase, BBLK)], wsems[k])

        def dispatch(kdyn, fn):
            for kk in range(_NBUF):
                @pl.when(kdyn == kk)
                def _go():
                    fn(kk)
            return None

        iota = jnp.arange(16, dtype=jnp.int32)
        iota17 = iota * 17

        def transpose_block(kk):
            @plsc.parallel_loop(0, nbb, step=1)
            def _tp(bb):
                b0 = bb * 16
                sc = tpscr.at[bb]
                for q in range(neg):
                    for r in range(16):
                        v = gbufs[kk][b0 + r, pl.ds(16 * q, 16)]
                        plsc.store_scatter(sc, [iota17 + r], v)
                    for c in range(16):
                        col = plsc.load_gather(sc, [iota + 17 * c])
                        tbufs[kk][16 * q + c, pl.ds(b0, 16)] = col

        for l0 in range(_INIT_AHEAD):
            idx_cp(l0, l0).start()
            for cp in init_cps(l0, l0):
                cp.start()
        for l0 in range(_GATHER_AHEAD):
            idx_cp(l0, l0).wait()
            idx_convert(l0)
            for cp in init_cps(l0, l0):
                cp.wait()
            gather_cp(l0, l0).start(add=True)

        def body(l, carry):
            @pl.when(l + _INIT_AHEAD < L)
            def _init_ahead():
                li = l + _INIT_AHEAD

                def go(kk):
                    idx_cp(li, kk).start()
                    for cp in init_cps(li, kk):
                        cp.start()

                dispatch(lax.rem(li, _NBUF), go)

            @pl.when(l + _GATHER_AHEAD < L)
            def _gather_ahead():
                lg = l + _GATHER_AHEAD

                def go(kk):
                    idx_cp(lg, kk).wait()
                    idx_convert(kk)
                    for cp in init_cps(lg, kk):
                        cp.wait()
                    gather_cp(lg, kk).start(add=True)

                dispatch(lax.rem(lg, _NBUF), go)

            def go(kk):
                gather_cp(l, kk).wait()

                @pl.when(l >= _NBUF)
                def _drain_wb():
                    wb_cp(l - _NBUF, kk).wait()

                transpose_block(kk)
                wb_cp(l, kk).start()

            dispatch(lax.rem(l, _NBUF), go)
            return carry

        lax.fori_loop(0, L, body, 0)
        for l in range(L - _NBUF, L):
            wb_cp(l, l % _NBUF).wait()

    out = emb_kernel(inputs_t, token_table, pos_table)
    return jnp.transpose(out, (2, 0, 1))


# speedup vs baseline: 1.1048x; 1.1048x over previous
import functools

import jax
import jax.numpy as jnp
from jax import lax
from jax.experimental import pallas as pl
from jax.experimental.pallas import tpu as pltpu
from jax.experimental.pallas import tpu_sc as plsc

_NUM_WORKERS = 32
_NBUF = 8
_INIT_AHEAD = 4
_GATHER_AHEAD = 2
_POSB_W = 32


def kernel(inputs, token_table, pos_table):
    B, L = inputs.shape
    V, E = token_table.shape
    BBLK = B // _NUM_WORKERS
    neg = E // 16

    inputs_t = jnp.swapaxes(inputs, 0, 1)

    mesh = plsc.VectorSubcoreMesh(core_axis_name="c", subcore_axis_name="s")

    @functools.partial(
        pl.kernel,
        mesh=mesh,
        compiler_params=pltpu.CompilerParams(use_tc_tiling_on_sc=False,
                                             needs_layout_passes=False),
        out_type=jax.ShapeDtypeStruct((L, B, E), jnp.float32),
        scratch_types=[
            pltpu.VMEM((L, BBLK), jnp.int32),
            pltpu.VMEM((L, E), jnp.float32),
            pltpu.VMEM_SHARED((L, _POSB_W, E), jnp.float32),
            [pltpu.VMEM((BBLK, E), jnp.float32)] * _NBUF,
            [pltpu.SemaphoreType.DMA] * _NBUF,
            [pltpu.SemaphoreType.DMA] * _NBUF,
            [pltpu.SemaphoreType.DMA] * _NBUF,
        ],
    )
    def emb_kernel(inputs_hbm, table_hbm, pos_hbm, out_hbm,
                   idx_v, pos_v, posb_sh, bufs, isems, gsems, wsems):
        sid = lax.axis_index("s")
        wid = sid * 2 + lax.axis_index("c")
        bbase = wid * BBLK

        pltpu.sync_copy(inputs_hbm.at[:, pl.ds(bbase, BBLK)], idx_v)
        pltpu.sync_copy(pos_hbm, pos_v)

        def build(l, carry):
            prow = [pos_v[l, pl.ds(16 * q, 16)] for q in range(neg)]

            @plsc.parallel_loop(0, _POSB_W, step=1)
            def _fill(j):
                for q in range(neg):
                    bufs[0][j, pl.ds(16 * q, 16)] = prow[q]

            pltpu.sync_copy(bufs[0].at[pl.ds(0, _POSB_W)], posb_sh.at[l])
            return carry

        lax.fori_loop((sid * L) // 16, ((sid + 1) * L) // 16, build, 0)
        plsc.subcore_barrier()

        def init_cps(l, k):
            return [
                pltpu.make_async_copy(
                    posb_sh.at[l],
                    bufs[k].at[pl.ds(i * _POSB_W, _POSB_W)], isems[k])
                for i in range(BBLK // _POSB_W)
            ]

        def gather_cp(l, k):
            return pltpu.make_async_copy(
                table_hbm.at[idx_v.at[l, :]], bufs[k], gsems[k])

        def wb_cp(l, k):
            return pltpu.make_async_copy(
                bufs[k], out_hbm.at[l, pl.ds(bbase, BBLK)], wsems[k])

        def dispatch(kdyn, fn):
            for kk in range(_NBUF):
                @pl.when(kdyn == kk)
                def _go():
                    fn(kk)
            return None

        for l0 in range(_INIT_AHEAD):
            for cp in init_cps(l0, l0):
                cp.start()
        for l0 in range(_GATHER_AHEAD):
            for cp in init_cps(l0, l0):
                cp.wait()
            gather_cp(l0, l0).start(add=True)

        def body(l, carry):
            @pl.when(l + _INIT_AHEAD < L)
            def _init_ahead():
                li = l + _INIT_AHEAD

                def go(kk):
                    @pl.when(li >= _NBUF)
                    def _drain_wb():
                        wb_cp(li - _NBUF, kk).wait()
                    for cp in init_cps(li, kk):
                        cp.start()

                dispatch(lax.rem(li, _NBUF), go)

            @pl.when(l + _GATHER_AHEAD < L)
            def _gather_ahead():
                lg = l + _GATHER_AHEAD

                def go(kk):
                    for cp in init_cps(lg, kk):
                        cp.wait()
                    gather_cp(lg, kk).start(add=True)

                dispatch(lax.rem(lg, _NBUF), go)

            def go(kk):
                gather_cp(l, kk).wait()
                wb_cp(l, kk).start()

            dispatch(lax.rem(l, _NBUF), go)
            return carry

        lax.fori_loop(0, L, body, 0)
        for l in range(L - _NBUF, L):
            wb_cp(l, l % _NBUF).wait()

    out = emb_kernel(inputs_t, token_table, pos_table)
    return jnp.transpose(out, (1, 0, 2))
